# baseline (speedup 1.0000x reference)
import functools

import jax
import jax.numpy as jnp
from jax import lax
from jax.experimental import pallas as pl
from jax.experimental.pallas import tpu as pltpu
from jax.experimental.pallas import tpu_sc as plsc

NC = 2
NS = 16
NW = NC * NS

E_DYN = 8
TOP_K = 2
D = 2048
F = 512
T = 2048
TILE = 256
P_DYN = ((T * TOP_K + E_DYN * (TILE - 1) + TILE - 1) // TILE) * TILE
N_TILES = P_DYN // TILE
NEG = -1e30
JITTER = 0.01


def _router_body(logits_ref, out_ref):
    lg = logits_ref[:]
    lane = lax.broadcasted_iota(jnp.int32, lg.shape, 1)
    s = jnp.where(lane < E_DYN, lg, NEG)

    thr1 = jnp.max(s, axis=-1, keepdims=True)
    sel1 = jnp.min(jnp.where(s == thr1, lane, 99), axis=-1, keepdims=True)
    factor1 = jnp.maximum(jnp.abs(s), jnp.abs(thr1))
    mask1 = (thr1 - s) / factor1 > 2.0 * JITTER
    gin1 = jnp.where(mask1, NEG, s)
    e1 = jnp.exp(gin1 - jnp.max(gin1, axis=-1, keepdims=True))
    gates1 = e1 / jnp.sum(e1, axis=-1, keepdims=True)
    mult1 = jnp.sum(jnp.where(lane == sel1, gates1, 0.0), axis=-1, keepdims=True)

    s2 = jnp.where(lane == sel1, NEG, s)
    thr2 = jnp.max(s2, axis=-1, keepdims=True)
    sel2 = jnp.min(jnp.where(s2 == thr2, lane, 99), axis=-1, keepdims=True)
    factor2 = jnp.maximum(jnp.abs(s), jnp.abs(thr2))
    mask2 = (thr2 - s) / factor2 > 2.0 * JITTER
    gin2 = jnp.where(mask2, NEG, s2)
    e2 = jnp.exp(gin2 - jnp.max(gin2, axis=-1, keepdims=True))
    gates2 = e2 / jnp.sum(e2, axis=-1, keepdims=True)
    mult2 = jnp.sum(jnp.where(lane == sel2, gates2, 0.0), axis=-1, keepdims=True)

    active = (lane == sel1) | (lane == sel2) | (lane == E_DYN)
    gwin = jnp.where(active, lg, NEG)
    eg = jnp.exp(gwin - jnp.max(gwin, axis=-1, keepdims=True))
    gw = eg / jnp.sum(eg, axis=-1, keepdims=True)
    gsum_dyn = jnp.sum(jnp.where(lane < E_DYN, gw, 0.0), axis=-1, keepdims=True)
    g_fix = jnp.sum(jnp.where(lane == E_DYN, gw, 0.0), axis=-1, keepdims=True)

    w0 = mult1 * gsum_dyn
    w1 = mult2 * gsum_dyn

    oh = ((lane == sel1) | (lane == sel2)).astype(jnp.float32)
    ti = lax.broadcasted_iota(jnp.int32, (T, T), 0)
    tj = lax.broadcasted_iota(jnp.int32, (T, T), 1)
    mstrict = (tj < ti).astype(jnp.float32)
    excl = jnp.dot(mstrict, oh, preferred_element_type=jnp.float32)
    counts = jnp.sum(oh, axis=0, keepdims=True)
    padded = jnp.floor((counts + (TILE - 1.0)) * (1.0 / TILE)) * TILE
    ei = lax.broadcasted_iota(jnp.int32, (16, 16), 0)
    ej = lax.broadcasted_iota(jnp.int32, (16, 16), 1)
    umat = (ei < ej).astype(jnp.float32)
    starts8 = jnp.dot(jnp.broadcast_to(padded, (8, 16)), umat,
                      preferred_element_type=jnp.float32)
    starts = starts8[0:1]
    pos1 = jnp.sum(jnp.where(lane == sel1, starts + excl, 0.0), axis=-1,
                   keepdims=True)
    pos2 = jnp.sum(jnp.where(lane == sel2, starts + excl, 0.0), axis=-1,
                   keepdims=True)

    out = (jnp.where(lane == 0, w0, 0.0)
           + jnp.where(lane == 1, w1, 0.0)
           + jnp.where(lane == 2, g_fix, 0.0)
           + jnp.where(lane == 3, sel1.astype(jnp.float32), 0.0)
           + jnp.where(lane == 4, sel2.astype(jnp.float32), 0.0)
           + jnp.where(lane == 5, pos1, 0.0)
           + jnp.where(lane == 6, pos2, 0.0))
    out_ref[:] = out[:, :8]


def _router(logits16):
    return pl.pallas_call(
        _router_body,
        out_shape=jax.ShapeDtypeStruct((T, 8), jnp.float32),
    )(logits16)


def _gather_rows(table, idx):
    P = idx.shape[0]
    rows_per_w = P // NW
    CH = 16
    chunks = rows_per_w // CH
    assert P % NW == 0 and rows_per_w % CH == 0

    @functools.partial(
        pl.kernel,
        out_type=jax.ShapeDtypeStruct((P, D), jnp.float32),
        mesh=plsc.VectorSubcoreMesh(core_axis_name="c", subcore_axis_name="s"),
        scratch_types=[
            pltpu.VMEM((rows_per_w,), jnp.int32),
            pltpu.VMEM((CH, D), jnp.float32),
            pltpu.VMEM((CH, D), jnp.float32),
            pltpu.SemaphoreType.DMA,
            pltpu.SemaphoreType.DMA,
        ],
    )
    def gk(x_hbm, idx_hbm, out_hbm, idx_v, buf0, buf1, sem0, sem1):
        wid = lax.axis_index("s") * NC + lax.axis_index("c")
        base = wid * rows_per_w
        pltpu.sync_copy(idx_hbm.at[pl.ds(base, rows_per_w)], idx_v)
        bufs = (buf0, buf1)
        sems = (sem0, sem1)
        copies = [
            pltpu.make_async_copy(
                x_hbm.at[idx_v.at[pl.ds(c * CH, CH)]], bufs[c % 2],
                sems[c % 2])
            for c in range(chunks)
        ]
        copies[0].start()
        for c in range(chunks):
            if c + 1 < chunks:
                copies[c + 1].start()
            copies[c].wait()
            pltpu.sync_copy(bufs[c % 2], out_hbm.at[pl.ds(base + c * CH, CH)])

    return gk(table, idx)


def _scatter_rows(x2d, pos1, pos2):
    tpw = T // NW
    CH = 16
    chunks = tpw // CH
    p1 = pos1.reshape(NW, chunks, CH)
    p2 = pos2.reshape(NW, chunks, CH)

    @functools.partial(
        pl.kernel,
        out_type=jax.ShapeDtypeStruct((P_DYN, D), jnp.float32),
        mesh=plsc.VectorSubcoreMesh(core_axis_name="c", subcore_axis_name="s"),
        scratch_types=[
            pltpu.VMEM((chunks, CH), jnp.int32),
            pltpu.VMEM((chunks, CH), jnp.int32),
            pltpu.VMEM((CH, D), jnp.float32),
            pltpu.VMEM((CH, D), jnp.float32),
            pltpu.SemaphoreType.DMA,
            pltpu.SemaphoreType.DMA,
            pltpu.SemaphoreType.DMA,
        ],
    )
    def sk(x_hbm, p1_hbm, p2_hbm, out_hbm, i1_v, i2_v, buf0, buf1, seml,
           sem1, sem2):
        wid = lax.axis_index("s") * NC + lax.axis_index("c")
        base = wid * tpw
        pltpu.sync_copy(p1_hbm.at[wid], i1_v)
        pltpu.sync_copy(p2_hbm.at[wid], i2_v)
        bufs = (buf0, buf1)
        loads = [
            pltpu.make_async_copy(
                x_hbm.at[pl.ds(base + c * CH, CH)], bufs[c % 2], seml)
            for c in range(chunks)
        ]
        loads[0].start()
        for c in range(chunks):
            if c + 1 < chunks:
                loads[c + 1].start()
            loads[c].wait()
            s1 = pltpu.make_async_copy(bufs[c % 2], out_hbm.at[i1_v.at[c]],
                                       sem1)
            s2 = pltpu.make_async_copy(bufs[c % 2], out_hbm.at[i2_v.at[c]],
                                       sem2)
            s1.start()
            s2.start()
            s1.wait()
            s2.wait()

    return sk(x2d, p1, p2)


def _grouped_body(te_ref, nt_ref, xs_ref, wg_ref, wu_ref, wd_ref, out_ref):
    @pl.when(pl.program_id(0) < nt_ref[0])
    def _():
        x = xs_ref[:]
        g = jnp.dot(x, wg_ref[0], preferred_element_type=jnp.float32)
        u = jnp.dot(x, wu_ref[0], preferred_element_type=jnp.float32)
        h = (g * jax.nn.sigmoid(g)) * u
        out_ref[:] = jnp.dot(h, wd_ref[0], preferred_element_type=jnp.float32)


def _grouped_ffn(xs, Wg, Wu, Wd, tile_expert, num_tiles):
    return pl.pallas_call(
        _grouped_body,
        grid_spec=pltpu.PrefetchScalarGridSpec(
            num_scalar_prefetch=2,
            grid=(N_TILES,),
            in_specs=[
                pl.BlockSpec((TILE, D), lambda i, te, nt: (i, 0)),
                pl.BlockSpec((1, D, F), lambda i, te, nt: (te[i], 0, 0)),
                pl.BlockSpec((1, D, F), lambda i, te, nt: (te[i], 0, 0)),
                pl.BlockSpec((1, F, D), lambda i, te, nt: (te[i], 0, 0)),
            ],
            out_specs=pl.BlockSpec((TILE, D), lambda i, te, nt: (i, 0)),
        ),
        out_shape=jax.ShapeDtypeStruct((P_DYN, D), jnp.float32),
    )(tile_expert, num_tiles, xs, Wg, Wu, Wd)


def _shared_body(x_ref, wg_ref, wu_ref, wd_ref, wrow_ref, out_ref):
    x = x_ref[:].astype(jnp.bfloat16)
    g = jnp.dot(x, wg_ref[0].astype(jnp.bfloat16),
                preferred_element_type=jnp.float32)
    u = jnp.dot(x, wu_ref[0].astype(jnp.bfloat16),
                preferred_element_type=jnp.float32)
    h = (g * jax.nn.sigmoid(g)) * u
    h = (h * wrow_ref[0, 0][:, None]).astype(jnp.bfloat16)
    out_ref[:] = jnp.dot(h, wd_ref[0].astype(jnp.bfloat16),
                         preferred_element_type=jnp.float32)


def _shared_ffn(x2d, Wg_sh, Wu_sh, Wd_sh, g_fix):
    H = T // 2
    wrow = g_fix.reshape(T // TILE, 1, TILE)
    halves = []
    for j in (0, 1):
        halves.append(pl.pallas_call(
            _shared_body,
            grid=(H // TILE,),
            in_specs=[
                pl.BlockSpec((TILE, D), lambda i: (i, 0)),
                pl.BlockSpec((1, D, F), lambda i: (0, 0, 0)),
                pl.BlockSpec((1, D, F), lambda i: (0, 0, 0)),
                pl.BlockSpec((1, F, D), lambda i: (0, 0, 0)),
                pl.BlockSpec((1, 1, TILE), lambda i: (i, 0, 0)),
            ],
            out_specs=pl.BlockSpec((TILE, D), lambda i: (i, 0)),
            out_shape=jax.ShapeDtypeStruct((H, D), jnp.float32),
        )(lax.slice_in_dim(x2d, j * H, (j + 1) * H, axis=0), Wg_sh, Wu_sh,
          Wd_sh, lax.slice_in_dim(wrow, j * (H // TILE), (j + 1) * (H // TILE),
                                  axis=0)))
    return jnp.concatenate(halves, axis=0)


def _combine_body(a_ref, b_ref, c_ref, w_ref, out_ref):
    w = w_ref[0]
    out_ref[:] = (w[0][:, None] * a_ref[:] + w[1][:, None] * b_ref[:]
                  + c_ref[:])


def _combine(ys, ysh, pos0, pos1, w0, w1):
    ye = _gather_rows(ys, jnp.concatenate([pos0, pos1]))
    wcomb = jnp.moveaxis(
        jnp.stack([w0, w1], axis=0).reshape(2, T // TILE, TILE), 0, 1)
    return pl.pallas_call(
        _combine_body,
        grid=(T // TILE,),
        in_specs=[
            pl.BlockSpec((TILE, D), lambda i: (i, 0)),
            pl.BlockSpec((TILE, D), lambda i: (T // TILE + i, 0)),
            pl.BlockSpec((TILE, D), lambda i: (i, 0)),
            pl.BlockSpec((1, 2, TILE), lambda i: (i, 0, 0)),
        ],
        out_specs=pl.BlockSpec((TILE, D), lambda i: (i, 0)),
        out_shape=jax.ShapeDtypeStruct((T, D), jnp.float32),
    )(ye, ye, ysh, wcomb)


def kernel(hidden_states, W_router, Wg_dyn, Wu_dyn, Wd_dyn, Wg_sh, Wu_sh,
           Wd_sh):
    B, S, _ = hidden_states.shape
    x2d = hidden_states.reshape(T, D)

    full_logits = x2d @ W_router
    logits16 = jnp.pad(full_logits, ((0, 0), (0, 16 - (E_DYN + 1))),
                       constant_values=NEG)
    r = _router(logits16)
    w0, w1, g_fix = r[:, 0], r[:, 1], r[:, 2]
    sel1 = r[:, 3].astype(jnp.int32)
    sel2 = r[:, 4].astype(jnp.int32)
    pos1 = r[:, 5].astype(jnp.int32)
    pos2 = r[:, 6].astype(jnp.int32)

    e_ar = jnp.arange(E_DYN, dtype=jnp.int32)
    counts = (jnp.sum((sel1[:, None] == e_ar).astype(jnp.int32), axis=0)
              + jnp.sum((sel2[:, None] == e_ar).astype(jnp.int32), axis=0))
    padded = ((counts + TILE - 1) // TILE) * TILE
    starts = jnp.concatenate([jnp.zeros((1,), jnp.int32),
                              jnp.cumsum(padded)[:-1].astype(jnp.int32)])
    num_tiles = (jnp.sum(padded) // TILE).astype(jnp.int32).reshape(1)
    tile_expert = jnp.clip(
        jnp.searchsorted(starts, jnp.arange(N_TILES, dtype=jnp.int32) * TILE,
                         side='right').astype(jnp.int32) - 1, 0, E_DYN - 1)

    xs = _scatter_rows(x2d, pos1, pos2)
    ys = _grouped_ffn(xs, Wg_dyn, Wu_dyn, Wd_dyn, tile_expert, num_tiles)
    ysh = _shared_ffn(x2d, Wg_sh, Wu_sh, Wd_sh, g_fix)
    out = _combine(ys, ysh, pos1, pos2, w0, w1)
    return out.reshape(B, S, D)


# speedup vs baseline: 1.1820x; 1.1820x over previous
import functools

import jax
import jax.numpy as jnp
from jax import lax
from jax.experimental import pallas as pl
from jax.experimental.pallas import tpu as pltpu
from jax.experimental.pallas import tpu_sc as plsc

NC = 2
NS = 16
NW = NC * NS

E_DYN = 8
TOP_K = 2
D = 2048
F = 512
T = 2048
TILE = 256
P_DYN = ((T * TOP_K + E_DYN * (TILE - 1) + TILE - 1) // TILE) * TILE
N_TILES = P_DYN // TILE
NEG = -1e30
JITTER = 0.01


def _router_body(logits_ref, out_ref):
    lg = logits_ref[:]
    lane = lax.broadcasted_iota(jnp.int32, lg.shape, 1)
    s = jnp.where(lane < E_DYN, lg, NEG)

    thr1 = jnp.max(s, axis=-1, keepdims=True)
    sel1 = jnp.min(jnp.where(s == thr1, lane, 99), axis=-1, keepdims=True)
    factor1 = jnp.maximum(jnp.abs(s), jnp.abs(thr1))
    mask1 = (thr1 - s) / factor1 > 2.0 * JITTER
    gin1 = jnp.where(mask1, NEG, s)
    e1 = jnp.exp(gin1 - jnp.max(gin1, axis=-1, keepdims=True))
    gates1 = e1 / jnp.sum(e1, axis=-1, keepdims=True)
    mult1 = jnp.sum(jnp.where(lane == sel1, gates1, 0.0), axis=-1, keepdims=True)

    s2 = jnp.where(lane == sel1, NEG, s)
    thr2 = jnp.max(s2, axis=-1, keepdims=True)
    sel2 = jnp.min(jnp.where(s2 == thr2, lane, 99), axis=-1, keepdims=True)
    factor2 = jnp.maximum(jnp.abs(s), jnp.abs(thr2))
    mask2 = (thr2 - s) / factor2 > 2.0 * JITTER
    gin2 = jnp.where(mask2, NEG, s2)
    e2 = jnp.exp(gin2 - jnp.max(gin2, axis=-1, keepdims=True))
    gates2 = e2 / jnp.sum(e2, axis=-1, keepdims=True)
    mult2 = jnp.sum(jnp.where(lane == sel2, gates2, 0.0), axis=-1, keepdims=True)

    active = (lane == sel1) | (lane == sel2) | (lane == E_DYN)
    gwin = jnp.where(active, lg, NEG)
    eg = jnp.exp(gwin - jnp.max(gwin, axis=-1, keepdims=True))
    gw = eg / jnp.sum(eg, axis=-1, keepdims=True)
    gsum_dyn = jnp.sum(jnp.where(lane < E_DYN, gw, 0.0), axis=-1, keepdims=True)
    g_fix = jnp.sum(jnp.where(lane == E_DYN, gw, 0.0), axis=-1, keepdims=True)

    w0 = mult1 * gsum_dyn
    w1 = mult2 * gsum_dyn

    oh = ((lane == sel1) | (lane == sel2)).astype(jnp.float32)
    ti = lax.broadcasted_iota(jnp.int32, (T, T), 0)
    tj = lax.broadcasted_iota(jnp.int32, (T, T), 1)
    mstrict = (tj < ti).astype(jnp.float32)
    excl = jnp.dot(mstrict, oh, preferred_element_type=jnp.float32)
    counts = jnp.sum(oh, axis=0, keepdims=True)
    padded = jnp.floor((counts + (TILE - 1.0)) * (1.0 / TILE)) * TILE
    ei = lax.broadcasted_iota(jnp.int32, (16, 16), 0)
    ej = lax.broadcasted_iota(jnp.int32, (16, 16), 1)
    umat = (ei < ej).astype(jnp.float32)
    starts8 = jnp.dot(jnp.broadcast_to(padded, (8, 16)), umat,
                      preferred_element_type=jnp.float32)
    starts = starts8[0:1]
    pos1 = jnp.sum(jnp.where(lane == sel1, starts + excl, 0.0), axis=-1,
                   keepdims=True)
    pos2 = jnp.sum(jnp.where(lane == sel2, starts + excl, 0.0), axis=-1,
                   keepdims=True)

    out = (jnp.where(lane == 0, w0, 0.0)
           + jnp.where(lane == 1, w1, 0.0)
           + jnp.where(lane == 2, g_fix, 0.0)
           + jnp.where(lane == 3, sel1.astype(jnp.float32), 0.0)
           + jnp.where(lane == 4, sel2.astype(jnp.float32), 0.0)
           + jnp.where(lane == 5, pos1, 0.0)
           + jnp.where(lane == 6, pos2, 0.0))
    out_ref[:] = out[:, :8]


def _router(logits16):
    return pl.pallas_call(
        _router_body,
        out_shape=jax.ShapeDtypeStruct((T, 8), jnp.float32),
    )(logits16)


def _gather_rows(table, idx):
    P = idx.shape[0]
    rows_per_w = P // NW
    CH = 16
    chunks = rows_per_w // CH
    assert P % NW == 0 and rows_per_w % CH == 0

    @functools.partial(
        pl.kernel,
        out_type=jax.ShapeDtypeStruct((P, D), jnp.float32),
        mesh=plsc.VectorSubcoreMesh(core_axis_name="c", subcore_axis_name="s"),
        scratch_types=[
            pltpu.VMEM((rows_per_w,), jnp.int32),
            pltpu.VMEM((CH, D), jnp.float32),
            pltpu.VMEM((CH, D), jnp.float32),
            pltpu.SemaphoreType.DMA,
            pltpu.SemaphoreType.DMA,
        ],
    )
    def gk(x_hbm, idx_hbm, out_hbm, idx_v, buf0, buf1, sem0, sem1):
        wid = lax.axis_index("s") * NC + lax.axis_index("c")
        base = wid * rows_per_w
        pltpu.sync_copy(idx_hbm.at[pl.ds(base, rows_per_w)], idx_v)
        bufs = (buf0, buf1)
        sems = (sem0, sem1)
        copies = [
            pltpu.make_async_copy(
                x_hbm.at[idx_v.at[pl.ds(c * CH, CH)]], bufs[c % 2],
                sems[c % 2])
            for c in range(chunks)
        ]
        copies[0].start()
        for c in range(chunks):
            if c + 1 < chunks:
                copies[c + 1].start()
            copies[c].wait()
            pltpu.sync_copy(bufs[c % 2], out_hbm.at[pl.ds(base + c * CH, CH)])

    return gk(table, idx)


def _scatter_rows(x2d, pos1, pos2):
    tpw = T // NW
    CH = 16
    chunks = tpw // CH
    p1 = pos1.reshape(NW, chunks, CH)
    p2 = pos2.reshape(NW, chunks, CH)

    @functools.partial(
        pl.kernel,
        out_type=jax.ShapeDtypeStruct((P_DYN, D), jnp.float32),
        mesh=plsc.VectorSubcoreMesh(core_axis_name="c", subcore_axis_name="s"),
        scratch_types=[
            pltpu.VMEM((chunks, CH), jnp.int32),
            pltpu.VMEM((chunks, CH), jnp.int32),
            pltpu.VMEM((CH, D), jnp.float32),
            pltpu.VMEM((CH, D), jnp.float32),
            pltpu.SemaphoreType.DMA,
            pltpu.SemaphoreType.DMA,
            pltpu.SemaphoreType.DMA,
        ],
    )
    def sk(x_hbm, p1_hbm, p2_hbm, out_hbm, i1_v, i2_v, buf0, buf1, seml,
           sem1, sem2):
        wid = lax.axis_index("s") * NC + lax.axis_index("c")
        base = wid * tpw
        pltpu.sync_copy(p1_hbm.at[wid], i1_v)
        pltpu.sync_copy(p2_hbm.at[wid], i2_v)
        bufs = (buf0, buf1)
        loads = [
            pltpu.make_async_copy(
                x_hbm.at[pl.ds(base + c * CH, CH)], bufs[c % 2], seml)
            for c in range(chunks)
        ]
        loads[0].start()
        for c in range(chunks):
            if c + 1 < chunks:
                loads[c + 1].start()
            loads[c].wait()
            s1 = pltpu.make_async_copy(bufs[c % 2], out_hbm.at[i1_v.at[c]],
                                       sem1)
            s2 = pltpu.make_async_copy(bufs[c % 2], out_hbm.at[i2_v.at[c]],
                                       sem2)
            s1.start()
            s2.start()
            s1.wait()
            s2.wait()

    return sk(x2d, p1, p2)


def _grouped_body(te_ref, nt_ref, xs_ref, wg_ref, wu_ref, wd_ref, out_ref):
    @pl.when(pl.program_id(0) < nt_ref[0])
    def _():
        x = xs_ref[:]
        g = jnp.dot(x, wg_ref[0], preferred_element_type=jnp.float32)
        u = jnp.dot(x, wu_ref[0], preferred_element_type=jnp.float32)
        h = (g * jax.nn.sigmoid(g)) * u
        out_ref[:] = jnp.dot(h, wd_ref[0], preferred_element_type=jnp.float32)


def _grouped_ffn(xs, Wg, Wu, Wd, tile_expert, num_tiles):
    return pl.pallas_call(
        _grouped_body,
        grid_spec=pltpu.PrefetchScalarGridSpec(
            num_scalar_prefetch=2,
            grid=(N_TILES,),
            in_specs=[
                pl.BlockSpec((TILE, D), lambda i, te, nt: (i, 0)),
                pl.BlockSpec((1, D, F), lambda i, te, nt: (te[i], 0, 0)),
                pl.BlockSpec((1, D, F), lambda i, te, nt: (te[i], 0, 0)),
                pl.BlockSpec((1, F, D), lambda i, te, nt: (te[i], 0, 0)),
            ],
            out_specs=pl.BlockSpec((TILE, D), lambda i, te, nt: (i, 0)),
        ),
        out_shape=jax.ShapeDtypeStruct((P_DYN, D), jnp.float32),
    )(tile_expert, num_tiles, xs, Wg, Wu, Wd)


def _shared_body(x_ref, wg_ref, wu_ref, wd_ref, wrow_ref, out_ref):
    x = x_ref[:].astype(jnp.bfloat16)
    g = jnp.dot(x, wg_ref[0].astype(jnp.bfloat16),
                preferred_element_type=jnp.float32)
    u = jnp.dot(x, wu_ref[0].astype(jnp.bfloat16),
                preferred_element_type=jnp.float32)
    h = (g * jax.nn.sigmoid(g)) * u
    h = (h * wrow_ref[0, 0][:, None]).astype(jnp.bfloat16)
    out_ref[:] = jnp.dot(h, wd_ref[0].astype(jnp.bfloat16),
                         preferred_element_type=jnp.float32)


def _shared_ffn(x2d, Wg_sh, Wu_sh, Wd_sh, g_fix):
    wrow = g_fix.reshape(T // TILE, 1, TILE)
    return pl.pallas_call(
        _shared_body,
        grid=(T // TILE,),
        in_specs=[
            pl.BlockSpec((TILE, D), lambda i: (i, 0)),
            pl.BlockSpec((1, D, F), lambda i: (0, 0, 0)),
            pl.BlockSpec((1, D, F), lambda i: (0, 0, 0)),
            pl.BlockSpec((1, F, D), lambda i: (0, 0, 0)),
            pl.BlockSpec((1, 1, TILE), lambda i: (i, 0, 0)),
        ],
        out_specs=pl.BlockSpec((TILE, D), lambda i: (i, 0)),
        out_shape=jax.ShapeDtypeStruct((T, D), jnp.float32),
    )(x2d, Wg_sh, Wu_sh, Wd_sh, wrow)


def _combine_body(a_ref, b_ref, c_ref, w_ref, out_ref):
    w = w_ref[0]
    out_ref[:] = (w[0][:, None] * a_ref[:] + w[1][:, None] * b_ref[:]
                  + c_ref[:])


def _combine(ys, ysh, pos0, pos1, w0, w1):
    ye = _gather_rows(ys, jnp.concatenate([pos0, pos1]))
    wcomb = jnp.moveaxis(
        jnp.stack([w0, w1], axis=0).reshape(2, T // TILE, TILE), 0, 1)
    return pl.pallas_call(
        _combine_body,
        grid=(T // TILE,),
        in_specs=[
            pl.BlockSpec((TILE, D), lambda i: (i, 0)),
            pl.BlockSpec((TILE, D), lambda i: (T // TILE + i, 0)),
            pl.BlockSpec((TILE, D), lambda i: (i, 0)),
            pl.BlockSpec((1, 2, TILE), lambda i: (i, 0, 0)),
        ],
        out_specs=pl.BlockSpec((TILE, D), lambda i: (i, 0)),
        out_shape=jax.ShapeDtypeStruct((T, D), jnp.float32),
    )(ye, ye, ysh, wcomb)


def kernel(hidden_states, W_router, Wg_dyn, Wu_dyn, Wd_dyn, Wg_sh, Wu_sh,
           Wd_sh):
    B, S, _ = hidden_states.shape
    x2d = hidden_states.reshape(T, D)

    full_logits = x2d @ W_router
    logits16 = jnp.pad(full_logits, ((0, 0), (0, 16 - (E_DYN + 1))),
                       constant_values=NEG)
    r = _router(logits16)
    w0, w1, g_fix = r[:, 0], r[:, 1], r[:, 2]
    sel1 = r[:, 3].astype(jnp.int32)
    sel2 = r[:, 4].astype(jnp.int32)
    pos1 = r[:, 5].astype(jnp.int32)
    pos2 = r[:, 6].astype(jnp.int32)

    e_ar = jnp.arange(E_DYN, dtype=jnp.int32)
    counts = (jnp.sum((sel1[:, None] == e_ar).astype(jnp.int32), axis=0)
              + jnp.sum((sel2[:, None] == e_ar).astype(jnp.int32), axis=0))
    padded = ((counts + TILE - 1) // TILE) * TILE
    starts = jnp.concatenate([jnp.zeros((1,), jnp.int32),
                              jnp.cumsum(padded)[:-1].astype(jnp.int32)])
    num_tiles = (jnp.sum(padded) // TILE).astype(jnp.int32).reshape(1)
    tile_expert = jnp.clip(
        jnp.searchsorted(starts, jnp.arange(N_TILES, dtype=jnp.int32) * TILE,
                         side='right').astype(jnp.int32) - 1, 0, E_DYN - 1)

    xs = _scatter_rows(x2d, pos1, pos2)
    ys = _grouped_ffn(xs, Wg_dyn, Wu_dyn, Wd_dyn, tile_expert, num_tiles)
    ysh = _shared_ffn(x2d, Wg_sh, Wu_sh, Wd_sh, g_fix)
    out = _combine(ys, ysh, pos1, pos2, w0, w1)
    return out.reshape(B, S, D)
